# baseline (speedup 1.0000x reference)
import numpy as np
import jax
import jax.numpy as jnp
from jax import lax
from jax.experimental import pallas as pl
from jax.experimental.pallas import tpu as pltpu
from jax.experimental.pallas import tpu_sc as plsc

_LENS = (4, 9, 6, 9, 4, 9, 6, 9)
_G = len(_LENS)
_MAXP = max(_LENS)
_T = 1024
_D = 768
_N = sum(_LENS)
_PPW = 2
_NWORK = _N // _PPW
_LANES = 16


def _static_maps():
    src = np.zeros((_G, _MAXP), dtype=np.int32)
    valid = np.zeros((_G, _MAXP), dtype=np.int32)
    off = 0
    for g, n in enumerate(_LENS):
        for j in range(_MAXP):
            src[g, j] = off + min(j, n - 1)
            valid[g, j] = 1 if j < n else 0
        off += n
    return src.reshape(-1), valid.reshape(-1)


_SRC, _VALID = _static_maps()


def _sc_pe_body(hw_hbm, idx_hbm, pe_hbm, idx_v, rows_v, pe_v, sem):
    wid = lax.axis_index("s") * 2 + lax.axis_index("c")

    @pl.when(wid < _NWORK)
    def _():
        base = wid * _PPW
        pltpu.sync_copy(idx_hbm, idx_v)
        pltpu.async_copy(hw_hbm.at[idx_v.at[wid]], rows_v, sem).wait()

        for p in range(_PPW):
            def chunk(i, _):
                sl = pl.ds(i * _LANES, _LANES)
                pe_v[p, sl] = (rows_v[p, sl] + rows_v[p + _PPW, sl]) * 0.5
                return 0
            lax.fori_loop(0, _D // _LANES, chunk, 0)

        pltpu.sync_copy(pe_v, pe_hbm.at[pl.ds(base, _PPW)])


def _sc_pe(hw_table, idx):
    mesh = plsc.VectorSubcoreMesh(core_axis_name="c", subcore_axis_name="s")
    return pl.kernel(
        _sc_pe_body,
        mesh=mesh,
        out_type=jax.ShapeDtypeStruct((_N, _D), jnp.float32),
        scratch_types=[
            pltpu.VMEM((_NWORK, 2 * _PPW), jnp.int32),
            pltpu.VMEM((2 * _PPW, _D), jnp.float32),
            pltpu.VMEM((_PPW, _D), jnp.float32),
            pltpu.SemaphoreType.DMA,
        ],
    )(hw_table, idx)


def _tc_body(src_ref, valid_ref, pe_ref, x_ref, out_ref, mask_ref):
    g = pl.program_id(0)
    j = pl.program_id(1)
    i = g * _MAXP + j
    v = valid_ref[i]
    s = src_ref[i]

    @pl.when(v == 1)
    def _():
        out_ref[...] = x_ref[...] + pe_ref[s, :][None, None, :]
        mask_ref[...] = jnp.ones_like(mask_ref)

    @pl.when(v == 0)
    def _():
        out_ref[...] = jnp.zeros_like(out_ref)
        mask_ref[...] = jnp.zeros_like(mask_ref)


def kernel(query_embeds, encoder_hidden_states, patch_positions,
           height_embedding, width_embedding):
    del query_embeds

    src = jnp.asarray(_SRC)
    valid = jnp.asarray(_VALID)
    pos = patch_positions.astype(jnp.int32)
    hidx = pos[:, 0].reshape(_NWORK, _PPW)
    widx = pos[:, 1].reshape(_NWORK, _PPW)
    hw_table = jnp.concatenate([height_embedding, width_embedding], axis=0)
    idx = jnp.concatenate([hidx, widx + height_embedding.shape[0]], axis=1)

    pe = _sc_pe(hw_table, idx)

    grid_spec = pltpu.PrefetchScalarGridSpec(
        num_scalar_prefetch=2,
        grid=(_G, _MAXP),
        in_specs=[
            pl.BlockSpec((_N, _D), lambda g, j, src, valid: (0, 0)),
            pl.BlockSpec((1, _T, _D),
                         lambda g, j, src, valid: (src[g * _MAXP + j], 0, 0)),
        ],
        out_specs=[
            pl.BlockSpec((1, _T, _D), lambda g, j, src, valid: (g, j, 0)),
            pl.BlockSpec((1, 1, 1, _T), lambda g, j, src, valid: (g, j, 0, 0)),
        ],
    )

    padded, mask4 = pl.pallas_call(
        _tc_body,
        grid_spec=grid_spec,
        out_shape=[
            jax.ShapeDtypeStruct((_G, _MAXP * _T, _D), jnp.float32),
            jax.ShapeDtypeStruct((_G, _MAXP, 1, _T), jnp.int32),
        ],
    )(src, valid, pe, encoder_hidden_states)

    return padded, mask4.reshape(_G, _MAXP * _T)


# speedup vs baseline: 1.0071x; 1.0071x over previous
import numpy as np
import jax
import jax.numpy as jnp
from jax import lax
from jax.experimental import pallas as pl
from jax.experimental.pallas import tpu as pltpu
from jax.experimental.pallas import tpu_sc as plsc

_LENS = (4, 9, 6, 9, 4, 9, 6, 9)
_G = len(_LENS)
_MAXP = max(_LENS)
_T = 1024
_D = 768
_N = sum(_LENS)
_PPW = 4
_NWORK = _N // _PPW
_LANES = 16


def _static_maps():
    src = np.zeros((_G, _MAXP), dtype=np.int32)
    valid = np.zeros((_G, _MAXP), dtype=np.int32)
    off = 0
    for g, n in enumerate(_LENS):
        for j in range(_MAXP):
            src[g, j] = off + min(j, n - 1)
            valid[g, j] = 1 if j < n else 0
        off += n
    return src.reshape(-1), valid.reshape(-1)


_SRC, _VALID = _static_maps()


def _sc_pe_body(hw_hbm, idx_hbm, pe_hbm, idx_v, rows_v, pe_v, sem):
    wid = lax.axis_index("s")

    @pl.when(wid < _NWORK)
    def _():
        base = wid * _PPW
        pltpu.sync_copy(idx_hbm, idx_v)
        pltpu.async_copy(hw_hbm.at[idx_v.at[wid]], rows_v, sem).wait()

        for p in range(_PPW):
            def chunk(i, _):
                sl = pl.ds(i * _LANES, _LANES)
                pe_v[p, sl] = (rows_v[p, sl] + rows_v[p + _PPW, sl]) * 0.5
                return 0
            lax.fori_loop(0, _D // _LANES, chunk, 0)

        pltpu.sync_copy(pe_v, pe_hbm.at[pl.ds(base, _PPW)])


def _sc_pe(hw_table, idx):
    mesh = plsc.VectorSubcoreMesh(core_axis_name="c", subcore_axis_name="s",
                                  num_cores=1)
    return pl.kernel(
        _sc_pe_body,
        mesh=mesh,
        out_type=jax.ShapeDtypeStruct((_N, _D), jnp.float32),
        scratch_types=[
            pltpu.VMEM((_NWORK, 2 * _PPW), jnp.int32),
            pltpu.VMEM((2 * _PPW, _D), jnp.float32),
            pltpu.VMEM((_PPW, _D), jnp.float32),
            pltpu.SemaphoreType.DMA,
        ],
    )(hw_table, idx)


def _tc_body(src_ref, valid_ref, pe_ref, x_ref, out_ref, mask_ref):
    g = pl.program_id(0)
    j = pl.program_id(1)
    i = g * _MAXP + j
    v = valid_ref[i]
    s = src_ref[i]

    @pl.when(v == 1)
    def _():
        out_ref[...] = x_ref[...] + pe_ref[s, :][None, None, :]
        mask_ref[...] = jnp.ones_like(mask_ref)

    @pl.when(v == 0)
    def _():
        out_ref[...] = jnp.zeros_like(out_ref)
        mask_ref[...] = jnp.zeros_like(mask_ref)


def kernel(query_embeds, encoder_hidden_states, patch_positions,
           height_embedding, width_embedding):
    del query_embeds

    src = jnp.asarray(_SRC)
    valid = jnp.asarray(_VALID)
    pos = patch_positions.astype(jnp.int32)
    hidx = pos[:, 0].reshape(_NWORK, _PPW)
    widx = pos[:, 1].reshape(_NWORK, _PPW)
    hw_table = jnp.concatenate([height_embedding, width_embedding], axis=0)
    idx = jnp.concatenate([hidx, widx + height_embedding.shape[0]], axis=1)

    pe = _sc_pe(hw_table, idx)

    grid_spec = pltpu.PrefetchScalarGridSpec(
        num_scalar_prefetch=2,
        grid=(_G, _MAXP),
        in_specs=[
            pl.BlockSpec((_N, _D), lambda g, j, src, valid: (0, 0)),
            pl.BlockSpec((1, _T, _D),
                         lambda g, j, src, valid: (src[g * _MAXP + j], 0, 0)),
        ],
        out_specs=[
            pl.BlockSpec((1, _T, _D), lambda g, j, src, valid: (g, j, 0)),
            pl.BlockSpec((1, 1, 1, _T), lambda g, j, src, valid: (g, j, 0, 0)),
        ],
    )

    padded, mask4 = pl.pallas_call(
        _tc_body,
        grid_spec=grid_spec,
        out_shape=[
            jax.ShapeDtypeStruct((_G, _MAXP * _T, _D), jnp.float32),
            jax.ShapeDtypeStruct((_G, _MAXP, 1, _T), jnp.int32),
        ],
    )(src, valid, pe, encoder_hidden_states)

    return padded, mask4.reshape(_G, _MAXP * _T)
